# baseline (speedup 1.0000x reference)
import functools

import jax
import jax.numpy as jnp
from jax import lax
from jax.experimental import pallas as pl
from jax.experimental.pallas import tpu as pltpu
from jax.experimental.pallas import tpu_sc as plsc

LANES = 16
N_PARTS = 2


@functools.lru_cache(maxsize=None)
def _make_gather_sum(n_nodes, d, b, fan, nc, ns, n_parts, part):
    nw = nc * ns
    bp = b // n_parts
    b_per_w = bp // nw
    sb = max(1, 128 // fan)
    while b_per_w % sb:
        sb -= 1
    n_chunks = b_per_w // sb
    idxw = sb * fan
    part_chunk0 = part * (bp // sb)

    mesh = plsc.VectorSubcoreMesh(core_axis_name="c", subcore_axis_name="s")

    nbuf = 2
    assert n_chunks % nbuf == 0 and n_chunks >= nbuf

    @functools.partial(
        pl.kernel,
        mesh=mesh,
        out_type=jax.ShapeDtypeStruct((bp, d), jnp.float32),
        scratch_types=[
            pltpu.VMEM((n_chunks, idxw), jnp.int32),
        ] + [pltpu.VMEM((idxw, d), jnp.float32) for _ in range(nbuf)] + [
            pltpu.VMEM((b_per_w, d), jnp.float32),
        ] + [pltpu.SemaphoreType.DMA for _ in range(nbuf + 1)],
    )
    def gather_sum(neigh_hbm, feats_hbm, out_hbm, idx_v, *rest):
        bufs = rest[:nbuf]
        acc_v = rest[nbuf]
        sems = rest[nbuf + 1:nbuf + 1 + nbuf]
        isem = rest[-1]
        cid = lax.axis_index("c")
        sid = lax.axis_index("s")
        wid = sid * nc + cid
        base = (part_chunk0 + wid * n_chunks) * idxw
        for k in range(n_chunks):
            pltpu.async_copy(
                neigh_hbm.at[pl.ds(base + k * idxw, idxw)], idx_v.at[k], isem)
        for k in range(n_chunks):
            pltpu.make_async_copy(
                neigh_hbm.at[pl.ds(0, idxw)], idx_v.at[0], isem).wait()
        for k in range(nbuf - 1):
            pltpu.async_copy(feats_hbm.at[idx_v.at[k]], bufs[k], sems[k])

        def acc_chunk(k, buf):
            def d_body(dst, carry2):
                for c in range(d // LANES):
                    cs = pl.ds(c * LANES, LANES)
                    v = buf[dst * fan, cs]
                    for j in range(1, fan):
                        v = v + buf[dst * fan + j, cs]
                    acc_v[k * sb + dst, cs] = v
                return carry2

            lax.fori_loop(0, sb, d_body, 0)

        def ring_body(i, carry):
            for u in range(nbuf):
                k = nbuf * i + u
                ahead = k + nbuf - 1
                nxt = (u + nbuf - 1) % nbuf

                @pl.when(ahead < n_chunks)
                def _():
                    pltpu.async_copy(
                        feats_hbm.at[idx_v.at[ahead]], bufs[nxt], sems[nxt])

                pltpu.make_async_copy(
                    feats_hbm.at[idx_v.at[0]], bufs[u], sems[u]).wait()
                acc_chunk(k, bufs[u])
            return carry

        lax.fori_loop(0, n_chunks // nbuf, ring_body, 0)
        pltpu.sync_copy(acc_v, out_hbm.at[pl.ds(wid * b_per_w, b_per_w)])

    return gather_sum


def _tc_body(inv, x_ref, s_ref, w_ref, b_ref, *refs):
    o_ref = refs[-1]
    x = x_ref[...]
    agg = (x + s_ref[...]) * inv
    h = jnp.concatenate([x, agg], axis=1)
    acc = lax.dot_general(
        h, w_ref[...], (((1,), (1,)), ((), ())),
        preferred_element_type=jnp.float32)
    o_ref[...] = jnp.maximum(acc + b_ref[...], 0.0)[:, None, :]


@functools.lru_cache(maxsize=None)
def _make_fused_linear(b, d, h, fan, bm, n_parts, part):
    bp = b // n_parts
    grid_n = bp // bm
    i0 = part * grid_n
    in_specs = [
        pl.BlockSpec((bm, d), lambda i: (i0 + i, 0)),
        pl.BlockSpec((bm, d), lambda i: (i, 0)),
        pl.BlockSpec((h, 2 * d), lambda i: (0, 0)),
        pl.BlockSpec((1, h), lambda i: (0, 0)),
    ]
    kwargs = {}
    if part > 0:
        in_specs.append(pl.BlockSpec((8, 1, 128), lambda i: (0, 0, 0)))
        kwargs["input_output_aliases"] = {4: 0}
    return pl.pallas_call(
        functools.partial(_tc_body, 1.0 / (fan + 1)),
        grid=(grid_n,),
        in_specs=in_specs,
        out_specs=pl.BlockSpec((bm, 1, h), lambda i: (i0 + i, 0, 0)),
        out_shape=jax.ShapeDtypeStruct((b, 1, h), jnp.float32),
        **kwargs,
    )


def kernel(x, nodes, feats, neigh0, neigh1, W0, b0, W1, b1):
    b, d = x.shape
    fan = neigh1.shape[1]
    n_nodes = feats.shape[0]
    h = W1.shape[0]

    info = plsc.get_sparse_core_info()
    nc, ns = info.num_cores, info.num_subcores
    nw = nc * ns
    b_per_w = b // (nw * N_PARTS)
    sb = max(1, 128 // fan)
    while b_per_w % sb:
        sb -= 1
    idxw = sb * fan

    neigh_rows = neigh1.reshape(b * fan)
    b1r = b1.reshape(1, h)

    sums = [
        _make_gather_sum(n_nodes, d, b, fan, nc, ns, N_PARTS, p)(
            neigh_rows, feats)
        for p in range(N_PARTS)
    ]
    out = None
    for p in range(N_PARTS):
        fused = _make_fused_linear(b, d, h, fan, 512, N_PARTS, p)
        args = (x, sums[p], W1, b1r) + (() if p == 0 else (out,))
        out = fused(*args)
    return out


# speedup vs baseline: 1.0477x; 1.0477x over previous
import functools

import jax
import jax.numpy as jnp
from jax import lax
from jax.experimental import pallas as pl
from jax.experimental.pallas import tpu as pltpu
from jax.experimental.pallas import tpu_sc as plsc

LANES = 16
N_PARTS = 1


@functools.lru_cache(maxsize=None)
def _make_gather_sum(n_nodes, d, b, fan, nc, ns, n_parts, part):
    nw = nc * ns
    bp = b // n_parts
    b_per_w = bp // nw
    sb = max(1, 128 // fan)
    while b_per_w % sb:
        sb -= 1
    n_chunks = b_per_w // sb
    idxw = sb * fan
    part_chunk0 = part * (bp // sb)

    mesh = plsc.VectorSubcoreMesh(core_axis_name="c", subcore_axis_name="s")

    nbuf = 2
    assert n_chunks % nbuf == 0 and n_chunks >= nbuf

    @functools.partial(
        pl.kernel,
        mesh=mesh,
        out_type=jax.ShapeDtypeStruct((bp, d), jnp.float32),
        scratch_types=[
            pltpu.VMEM((n_chunks, idxw), jnp.int32),
        ] + [pltpu.VMEM((idxw, d), jnp.float32) for _ in range(nbuf)] + [
            pltpu.VMEM((b_per_w, d), jnp.float32),
        ] + [pltpu.SemaphoreType.DMA for _ in range(nbuf + 1)],
    )
    def gather_sum(neigh_hbm, feats_hbm, out_hbm, idx_v, *rest):
        bufs = rest[:nbuf]
        acc_v = rest[nbuf]
        sems = rest[nbuf + 1:nbuf + 1 + nbuf]
        isem = rest[-1]
        cid = lax.axis_index("c")
        sid = lax.axis_index("s")
        wid = sid * nc + cid
        base = (part_chunk0 + wid * n_chunks) * idxw
        for k in range(n_chunks):
            pltpu.async_copy(
                neigh_hbm.at[pl.ds(base + k * idxw, idxw)], idx_v.at[k], isem)
        for k in range(n_chunks):
            pltpu.make_async_copy(
                neigh_hbm.at[pl.ds(0, idxw)], idx_v.at[0], isem).wait()
        for k in range(nbuf - 1):
            pltpu.async_copy(feats_hbm.at[idx_v.at[k]], bufs[k], sems[k])

        def acc_chunk(k, buf):
            @plsc.parallel_loop(0, sb, unroll=2)
            def d_body(dst):
                for c in range(d // LANES):
                    cs = pl.ds(c * LANES, LANES)
                    v = buf[dst * fan, cs]
                    for j in range(1, fan):
                        v = v + buf[dst * fan + j, cs]
                    acc_v[k * sb + dst, cs] = v

        def ring_body(i, carry):
            for u in range(nbuf):
                k = nbuf * i + u
                ahead = k + nbuf - 1
                nxt = (u + nbuf - 1) % nbuf

                @pl.when(ahead < n_chunks)
                def _():
                    pltpu.async_copy(
                        feats_hbm.at[idx_v.at[ahead]], bufs[nxt], sems[nxt])

                pltpu.make_async_copy(
                    feats_hbm.at[idx_v.at[0]], bufs[u], sems[u]).wait()
                acc_chunk(k, bufs[u])
            return carry

        lax.fori_loop(0, n_chunks // nbuf, ring_body, 0)
        pltpu.sync_copy(acc_v, out_hbm.at[pl.ds(wid * b_per_w, b_per_w)])

    return gather_sum


def _tc_body(inv, x_ref, s_ref, w_ref, b_ref, *refs):
    o_ref = refs[-1]
    x = x_ref[...]
    agg = (x + s_ref[...]) * inv
    h = jnp.concatenate([x, agg], axis=1)
    acc = lax.dot_general(
        h, w_ref[...], (((1,), (1,)), ((), ())),
        preferred_element_type=jnp.float32)
    o_ref[...] = jnp.maximum(acc + b_ref[...], 0.0)[:, None, :]


@functools.lru_cache(maxsize=None)
def _make_fused_linear(b, d, h, fan, bm, n_parts, part):
    bp = b // n_parts
    grid_n = bp // bm
    i0 = part * grid_n
    in_specs = [
        pl.BlockSpec((bm, d), lambda i: (i0 + i, 0)),
        pl.BlockSpec((bm, d), lambda i: (i, 0)),
        pl.BlockSpec((h, 2 * d), lambda i: (0, 0)),
        pl.BlockSpec((1, h), lambda i: (0, 0)),
    ]
    kwargs = {}
    if part > 0:
        in_specs.append(pl.BlockSpec((8, 1, 128), lambda i: (0, 0, 0)))
        kwargs["input_output_aliases"] = {4: 0}
    return pl.pallas_call(
        functools.partial(_tc_body, 1.0 / (fan + 1)),
        grid=(grid_n,),
        in_specs=in_specs,
        out_specs=pl.BlockSpec((bm, 1, h), lambda i: (i0 + i, 0, 0)),
        out_shape=jax.ShapeDtypeStruct((b, 1, h), jnp.float32),
        **kwargs,
    )


def kernel(x, nodes, feats, neigh0, neigh1, W0, b0, W1, b1):
    b, d = x.shape
    fan = neigh1.shape[1]
    n_nodes = feats.shape[0]
    h = W1.shape[0]

    info = plsc.get_sparse_core_info()
    nc, ns = info.num_cores, info.num_subcores
    nw = nc * ns
    b_per_w = b // (nw * N_PARTS)
    sb = max(1, 128 // fan)
    while b_per_w % sb:
        sb -= 1
    idxw = sb * fan

    neigh_rows = neigh1.reshape(b * fan)
    b1r = b1.reshape(1, h)

    sums = [
        _make_gather_sum(n_nodes, d, b, fan, nc, ns, N_PARTS, p)(
            neigh_rows, feats)
        for p in range(N_PARTS)
    ]
    out = None
    for p in range(N_PARTS):
        fused = _make_fused_linear(b, d, h, fan, 512, N_PARTS, p)
        args = (x, sums[p], W1, b1r) + (() if p == 0 else (out,))
        out = fused(*args)
    return out


# speedup vs baseline: 1.1620x; 1.1091x over previous
import functools

import jax
import jax.numpy as jnp
from jax import lax
from jax.experimental import pallas as pl
from jax.experimental.pallas import tpu as pltpu
from jax.experimental.pallas import tpu_sc as plsc

LANES = 16
N_PARTS = 1


@functools.lru_cache(maxsize=None)
def _make_gather_sum(n_nodes, d, b, fan, nc, ns, n_parts, part):
    nw = nc * ns
    bp = b // n_parts
    b_per_w = bp // nw
    sb = max(1, 128 // fan)
    while b_per_w % sb:
        sb -= 1
    n_chunks = b_per_w // sb
    idxw = sb * fan
    part_chunk0 = part * (bp // sb)

    mesh = plsc.VectorSubcoreMesh(core_axis_name="c", subcore_axis_name="s")

    nbuf = 2
    assert n_chunks % nbuf == 0 and n_chunks >= nbuf

    @functools.partial(
        pl.kernel,
        mesh=mesh,
        out_type=jax.ShapeDtypeStruct((bp, d), jnp.float32),
        scratch_types=[
            pltpu.VMEM((n_chunks, idxw), jnp.int32),
        ] + [pltpu.VMEM((idxw, d), jnp.float32) for _ in range(nbuf)] + [
            pltpu.VMEM((b_per_w, d), jnp.float32),
        ] + [pltpu.SemaphoreType.DMA for _ in range(nbuf + 1)],
    )
    def gather_sum(neigh_hbm, feats_hbm, out_hbm, idx_v, *rest):
        bufs = rest[:nbuf]
        acc_v = rest[nbuf]
        sems = rest[nbuf + 1:nbuf + 1 + nbuf]
        isem = rest[-1]
        cid = lax.axis_index("c")
        sid = lax.axis_index("s")
        wid = sid * nc + cid
        base = (part_chunk0 + wid * n_chunks) * idxw
        for k in range(n_chunks):
            pltpu.async_copy(
                neigh_hbm.at[pl.ds(base + k * idxw, idxw)], idx_v.at[k], isem)
        for k in range(n_chunks):
            pltpu.make_async_copy(
                neigh_hbm.at[pl.ds(0, idxw)], idx_v.at[0], isem).wait()
        for k in range(nbuf - 1):
            pltpu.async_copy(feats_hbm.at[idx_v.at[k]], bufs[k], sems[k])

        def acc_chunk(k, buf):
            @plsc.parallel_loop(0, sb)
            def d_body(dst):
                for c in range(d // LANES):
                    cs = pl.ds(c * LANES, LANES)
                    v = buf[dst * fan, cs]
                    for j in range(1, fan):
                        v = v + buf[dst * fan + j, cs]
                    acc_v[k * sb + dst, cs] = v

        def ring_body(i, carry):
            for u in range(nbuf):
                k = nbuf * i + u
                ahead = k + nbuf - 1
                nxt = (u + nbuf - 1) % nbuf

                @pl.when(ahead < n_chunks)
                def _():
                    pltpu.async_copy(
                        feats_hbm.at[idx_v.at[ahead]], bufs[nxt], sems[nxt])

                pltpu.make_async_copy(
                    feats_hbm.at[idx_v.at[0]], bufs[u], sems[u]).wait()
                acc_chunk(k, bufs[u])
            return carry

        lax.fori_loop(0, n_chunks // nbuf, ring_body, 0)
        pltpu.sync_copy(acc_v, out_hbm.at[pl.ds(wid * b_per_w, b_per_w)])

    return gather_sum


def _tc_body(inv, x_ref, s_ref, w_ref, b_ref, *refs):
    o_ref = refs[-1]
    x = x_ref[...]
    agg = (x + s_ref[...]) * inv
    h = jnp.concatenate([x, agg], axis=1)
    acc = lax.dot_general(
        h, w_ref[...], (((1,), (1,)), ((), ())),
        preferred_element_type=jnp.float32)
    o_ref[...] = jnp.maximum(acc + b_ref[...], 0.0)[:, None, :]


@functools.lru_cache(maxsize=None)
def _make_fused_linear(b, d, h, fan, bm, n_parts, part):
    bp = b // n_parts
    grid_n = bp // bm
    i0 = part * grid_n
    in_specs = [
        pl.BlockSpec((bm, d), lambda i: (i0 + i, 0)),
        pl.BlockSpec((bm, d), lambda i: (i, 0)),
        pl.BlockSpec((h, 2 * d), lambda i: (0, 0)),
        pl.BlockSpec((1, h), lambda i: (0, 0)),
    ]
    kwargs = {}
    if part > 0:
        in_specs.append(pl.BlockSpec((8, 1, 128), lambda i: (0, 0, 0)))
        kwargs["input_output_aliases"] = {4: 0}
    return pl.pallas_call(
        functools.partial(_tc_body, 1.0 / (fan + 1)),
        grid=(grid_n,),
        in_specs=in_specs,
        out_specs=pl.BlockSpec((bm, 1, h), lambda i: (i0 + i, 0, 0)),
        out_shape=jax.ShapeDtypeStruct((b, 1, h), jnp.float32),
        **kwargs,
    )


def kernel(x, nodes, feats, neigh0, neigh1, W0, b0, W1, b1):
    b, d = x.shape
    fan = neigh1.shape[1]
    n_nodes = feats.shape[0]
    h = W1.shape[0]

    info = plsc.get_sparse_core_info()
    nc, ns = info.num_cores, info.num_subcores
    nw = nc * ns
    b_per_w = b // (nw * N_PARTS)
    sb = max(1, 128 // fan)
    while b_per_w % sb:
        sb -= 1
    idxw = sb * fan

    neigh_rows = neigh1.reshape(b * fan)
    b1r = b1.reshape(1, h)

    sums = [
        _make_gather_sum(n_nodes, d, b, fan, nc, ns, N_PARTS, p)(
            neigh_rows, feats)
        for p in range(N_PARTS)
    ]
    out = None
    for p in range(N_PARTS):
        fused = _make_fused_linear(b, d, h, fan, 512, N_PARTS, p)
        args = (x, sums[p], W1, b1r) + (() if p == 0 else (out,))
        out = fused(*args)
    return out


# speedup vs baseline: 1.1818x; 1.0170x over previous
import functools

import jax
import jax.numpy as jnp
from jax import lax
from jax.experimental import pallas as pl
from jax.experimental.pallas import tpu as pltpu
from jax.experimental.pallas import tpu_sc as plsc

LANES = 16
N_PARTS = 1


@functools.lru_cache(maxsize=None)
def _make_gather_sum(n_nodes, d, b, fan, nc, ns, n_parts, part):
    nw = nc * ns
    bp = b // n_parts
    b_per_w = bp // nw
    sb = max(1, 128 // fan)
    while b_per_w % sb:
        sb -= 1
    n_chunks = b_per_w // sb
    idxw = sb * fan
    part_chunk0 = part * (bp // sb)

    mesh = plsc.VectorSubcoreMesh(core_axis_name="c", subcore_axis_name="s")

    nbuf = 2
    assert n_chunks % nbuf == 0 and n_chunks >= nbuf

    @functools.partial(
        pl.kernel,
        mesh=mesh,
        out_type=jax.ShapeDtypeStruct((bp, d), jnp.float32),
        scratch_types=[
            pltpu.VMEM((n_chunks, idxw), jnp.int32),
        ] + [pltpu.VMEM((idxw, d), jnp.float32) for _ in range(nbuf)] + [
            pltpu.VMEM((b_per_w, d), jnp.float32),
        ] + [pltpu.SemaphoreType.DMA for _ in range(nbuf + 2)],
    )
    def gather_sum(neigh_hbm, feats_hbm, out_hbm, idx_v, *rest):
        bufs = rest[:nbuf]
        acc_v = rest[nbuf]
        sems = rest[nbuf + 1:nbuf + 1 + nbuf]
        isem = rest[-2]
        osem = rest[-1]
        cid = lax.axis_index("c")
        sid = lax.axis_index("s")
        wid = sid * nc + cid
        base = (part_chunk0 + wid * n_chunks) * idxw
        for k in range(n_chunks):
            pltpu.async_copy(
                neigh_hbm.at[pl.ds(base + k * idxw, idxw)], idx_v.at[k], isem)
        for k in range(n_chunks):
            pltpu.make_async_copy(
                neigh_hbm.at[pl.ds(0, idxw)], idx_v.at[0], isem).wait()
        for k in range(nbuf - 1):
            pltpu.async_copy(feats_hbm.at[idx_v.at[k]], bufs[k], sems[k])

        def acc_chunk(k, buf):
            @plsc.parallel_loop(0, sb)
            def d_body(dst):
                for c in range(d // LANES):
                    cs = pl.ds(c * LANES, LANES)
                    v = buf[dst * fan, cs]
                    for j in range(1, fan):
                        v = v + buf[dst * fan + j, cs]
                    acc_v[k * sb + dst, cs] = v

        def ring_body(i, carry):
            for u in range(nbuf):
                k = nbuf * i + u
                ahead = k + nbuf - 1
                nxt = (u + nbuf - 1) % nbuf

                @pl.when(ahead < n_chunks)
                def _():
                    pltpu.async_copy(
                        feats_hbm.at[idx_v.at[ahead]], bufs[nxt], sems[nxt])

                pltpu.make_async_copy(
                    feats_hbm.at[idx_v.at[0]], bufs[u], sems[u]).wait()
                acc_chunk(k, bufs[u])
                pltpu.async_copy(
                    acc_v.at[pl.ds(k * sb, sb)],
                    out_hbm.at[pl.ds(wid * b_per_w + k * sb, sb)], osem)
            return carry

        lax.fori_loop(0, n_chunks // nbuf, ring_body, 0)
        for k in range(n_chunks):
            pltpu.make_async_copy(
                acc_v.at[pl.ds(0, sb)],
                out_hbm.at[pl.ds(0, sb)], osem).wait()

    return gather_sum


def _tc_body(inv, x_ref, s_ref, w_ref, b_ref, *refs):
    o_ref = refs[-1]
    x = x_ref[...]
    agg = (x + s_ref[...]) * inv
    h = jnp.concatenate([x, agg], axis=1)
    acc = lax.dot_general(
        h, w_ref[...], (((1,), (1,)), ((), ())),
        preferred_element_type=jnp.float32)
    o_ref[...] = jnp.maximum(acc + b_ref[...], 0.0)[:, None, :]


@functools.lru_cache(maxsize=None)
def _make_fused_linear(b, d, h, fan, bm, n_parts, part):
    bp = b // n_parts
    grid_n = bp // bm
    i0 = part * grid_n
    in_specs = [
        pl.BlockSpec((bm, d), lambda i: (i0 + i, 0)),
        pl.BlockSpec((bm, d), lambda i: (i, 0)),
        pl.BlockSpec((h, 2 * d), lambda i: (0, 0)),
        pl.BlockSpec((1, h), lambda i: (0, 0)),
    ]
    kwargs = {}
    if part > 0:
        in_specs.append(pl.BlockSpec((8, 1, 128), lambda i: (0, 0, 0)))
        kwargs["input_output_aliases"] = {4: 0}
    return pl.pallas_call(
        functools.partial(_tc_body, 1.0 / (fan + 1)),
        grid=(grid_n,),
        in_specs=in_specs,
        out_specs=pl.BlockSpec((bm, 1, h), lambda i: (i0 + i, 0, 0)),
        out_shape=jax.ShapeDtypeStruct((b, 1, h), jnp.float32),
        **kwargs,
    )


def kernel(x, nodes, feats, neigh0, neigh1, W0, b0, W1, b1):
    b, d = x.shape
    fan = neigh1.shape[1]
    n_nodes = feats.shape[0]
    h = W1.shape[0]

    info = plsc.get_sparse_core_info()
    nc, ns = info.num_cores, info.num_subcores
    nw = nc * ns
    b_per_w = b // (nw * N_PARTS)
    sb = max(1, 128 // fan)
    while b_per_w % sb:
        sb -= 1
    idxw = sb * fan

    neigh_rows = neigh1.reshape(b * fan)
    b1r = b1.reshape(1, h)

    sums = [
        _make_gather_sum(n_nodes, d, b, fan, nc, ns, N_PARTS, p)(
            neigh_rows, feats)
        for p in range(N_PARTS)
    ]
    out = None
    for p in range(N_PARTS):
        fused = _make_fused_linear(b, d, h, fan, 512, N_PARTS, p)
        args = (x, sums[p], W1, b1r) + (() if p == 0 else (out,))
        out = fused(*args)
    return out
